# speedup vs baseline: 1.2774x; 1.2774x over previous
import functools

import jax
import jax.numpy as jnp
from jax import lax
from jax.experimental import pallas as pl
from jax.experimental.pallas import tpu as pltpu
from jax.experimental.pallas import tpu_sc as plsc

HIDDEN = 128
BATCH = 4
SEQ = 2048

NC, NS, L = 2, 16, 16
NW = NC * NS
N = BATCH * SEQ
RPW = N // NW
CHUNK = 128
NCHUNK = RPW // CHUNK
LANES_PER_ROW = HIDDEN // L


@functools.partial(
    pl.kernel,
    out_type=jax.ShapeDtypeStruct((N, HIDDEN), jnp.float32),
    mesh=plsc.VectorSubcoreMesh(core_axis_name="c", subcore_axis_name="s"),
    scratch_types=[
        pltpu.VMEM((RPW,), jnp.int32),
        pltpu.VMEM((RPW, HIDDEN), jnp.float32),
        pltpu.VMEM((RPW, HIDDEN), jnp.float32),
        pltpu.SemaphoreType.DMA,
    ],
)
def _embed_sum(ids_hbm, wtab_hbm, ptab_hbm, out_hbm, idx_v, rows_v, pos_v, sem):
    wid = lax.axis_index("s") * NC + lax.axis_index("c")
    base = wid * RPW
    pos_base = lax.rem(base, SEQ)

    pltpu.sync_copy(ids_hbm.at[pl.ds(base, RPW)], idx_v)

    gathers = []
    for ci in range(NCHUNK):
        gathers.append(
            pltpu.async_copy(
                wtab_hbm.at[idx_v.at[pl.ds(ci * CHUNK, CHUNK)]],
                rows_v.at[pl.ds(ci * CHUNK, CHUNK), :],
                sem,
            )
        )

    pltpu.sync_copy(ptab_hbm.at[pl.ds(pos_base, RPW)], pos_v)

    for g in gathers:
        g.wait()

    def add_row(i, _):
        for j in range(LANES_PER_ROW):
            sl = pl.ds(j * L, L)
            plsc.addupdate(rows_v.at[i, sl], pos_v[i, sl])
        return _

    lax.fori_loop(0, RPW, add_row, None)

    pltpu.sync_copy(rows_v, out_hbm.at[pl.ds(base, RPW)])


def kernel(input_ids, word_embeddings, position_embeddings):
    ids = input_ids.astype(jnp.int32).reshape(-1)
    out = _embed_sum(ids, word_embeddings, position_embeddings)
    return out.reshape(BATCH, SEQ, HIDDEN)


# speedup vs baseline: 1.3519x; 1.0584x over previous
import functools

import jax
import jax.numpy as jnp
from jax import lax
from jax.experimental import pallas as pl
from jax.experimental.pallas import tpu as pltpu
from jax.experimental.pallas import tpu_sc as plsc

HIDDEN = 128
BATCH = 4
SEQ = 2048

NC, NS, L = 2, 16, 16
NW = NC * NS
N = BATCH * SEQ
RPW = N // NW
CHUNK = 128
NCHUNK = RPW // CHUNK
LANES_PER_ROW = HIDDEN // L


@functools.partial(
    pl.kernel,
    out_type=jax.ShapeDtypeStruct((N, HIDDEN), jnp.float32),
    mesh=plsc.VectorSubcoreMesh(core_axis_name="c", subcore_axis_name="s"),
    scratch_types=[
        pltpu.VMEM((RPW,), jnp.int32),
        pltpu.VMEM((RPW, HIDDEN), jnp.float32),
        pltpu.VMEM((RPW, HIDDEN), jnp.float32),
        pltpu.SemaphoreType.DMA,
    ],
)
def _embed_sum(ids_hbm, wtab_hbm, ptab_hbm, out_hbm, idx_v, rows_v, pos_v, sem):
    wid = lax.axis_index("s") * NC + lax.axis_index("c")
    base = wid * RPW
    pos_base = lax.rem(base, SEQ)

    pltpu.sync_copy(ids_hbm.at[pl.ds(base, RPW)], idx_v)
    pltpu.sync_copy(ptab_hbm.at[pl.ds(pos_base, RPW)], rows_v)

    gathers = []
    for ci in range(NCHUNK):
        gathers.append(
            pltpu.async_copy(
                wtab_hbm.at[idx_v.at[pl.ds(ci * CHUNK, CHUNK)]],
                rows_v.at[pl.ds(ci * CHUNK, CHUNK), :],
                sem,
                add=True,
            )
        )
    for g in gathers:
        g.wait()

    pltpu.sync_copy(rows_v, out_hbm.at[pl.ds(base, RPW)])


def kernel(input_ids, word_embeddings, position_embeddings):
    ids = input_ids.astype(jnp.int32).reshape(-1)
    out = _embed_sum(ids, word_embeddings, position_embeddings)
    return out.reshape(BATCH, SEQ, HIDDEN)


# speedup vs baseline: 1.3684x; 1.0121x over previous
import functools

import jax
import jax.numpy as jnp
from jax import lax
from jax.experimental import pallas as pl
from jax.experimental.pallas import tpu as pltpu
from jax.experimental.pallas import tpu_sc as plsc

HIDDEN = 128
BATCH = 4
SEQ = 2048

NC, NS, L = 2, 16, 16
NW = NC * NS
N = BATCH * SEQ
RPW = N // NW
CHUNK = 64
NCHUNK = RPW // CHUNK


@functools.partial(
    pl.kernel,
    out_type=jax.ShapeDtypeStruct((N, HIDDEN), jnp.float32),
    mesh=plsc.VectorSubcoreMesh(core_axis_name="c", subcore_axis_name="s"),
    scratch_types=[
        pltpu.VMEM((RPW,), jnp.int32),
        pltpu.VMEM((RPW, HIDDEN), jnp.float32),
        [pltpu.SemaphoreType.DMA] * NCHUNK,
        [pltpu.SemaphoreType.DMA] * NCHUNK,
        pltpu.SemaphoreType.DMA,
    ],
)
def _embed_sum(ids_hbm, wtab_hbm, ptab_hbm, out_hbm, idx_v, rows_v,
               sem_pos, sem_g, sem_out):
    wid = lax.axis_index("s") * NC + lax.axis_index("c")
    base = wid * RPW
    pos_base = lax.rem(base, SEQ)

    pltpu.sync_copy(ids_hbm.at[pl.ds(base, RPW)], idx_v)

    pos_copies = []
    for ci in range(NCHUNK):
        sl = pl.ds(ci * CHUNK, CHUNK)
        pos_copies.append(
            pltpu.async_copy(
                ptab_hbm.at[pl.ds(pos_base + ci * CHUNK, CHUNK)],
                rows_v.at[sl, :],
                sem_pos[ci],
            )
        )

    gathers = []
    for ci in range(NCHUNK):
        sl = pl.ds(ci * CHUNK, CHUNK)
        pos_copies[ci].wait()
        gathers.append(
            pltpu.async_copy(
                wtab_hbm.at[idx_v.at[sl]],
                rows_v.at[sl, :],
                sem_g[ci],
                add=True,
            )
        )

    outs = []
    for ci in range(NCHUNK):
        sl = pl.ds(ci * CHUNK, CHUNK)
        gathers[ci].wait()
        outs.append(
            pltpu.async_copy(
                rows_v.at[sl, :],
                out_hbm.at[pl.ds(base + ci * CHUNK, CHUNK)],
                sem_out,
            )
        )
    for o in outs:
        o.wait()


def kernel(input_ids, word_embeddings, position_embeddings):
    ids = input_ids.astype(jnp.int32).reshape(-1)
    out = _embed_sum(ids, word_embeddings, position_embeddings)
    return out.reshape(BATCH, SEQ, HIDDEN)


# speedup vs baseline: 1.4057x; 1.0273x over previous
import functools

import jax
import jax.numpy as jnp
from jax import lax
from jax.experimental import pallas as pl
from jax.experimental.pallas import tpu as pltpu
from jax.experimental.pallas import tpu_sc as plsc

HIDDEN = 128
BATCH = 4
SEQ = 2048

NC, NS, L = 2, 16, 16
NW = NC * NS
N = BATCH * SEQ
PPW = SEQ // NW
RPW = BATCH * PPW


@functools.partial(
    pl.kernel,
    out_type=jax.ShapeDtypeStruct((N, HIDDEN), jnp.float32),
    mesh=plsc.VectorSubcoreMesh(core_axis_name="c", subcore_axis_name="s"),
    scratch_types=[
        pltpu.VMEM((RPW,), jnp.int32),
        pltpu.VMEM((RPW, HIDDEN), jnp.float32),
        pltpu.VMEM_SHARED((NS, PPW, HIDDEN), jnp.float32),
        pltpu.SemaphoreType.DMA,
        [pltpu.SemaphoreType.DMA] * BATCH,
        [pltpu.SemaphoreType.DMA] * BATCH,
        pltpu.SemaphoreType.DMA,
    ],
)
def _embed_sum(ids_hbm, wtab_hbm, ptab_hbm, out_hbm, idx_v, rows_v, pos_sh,
               sem_i, sem_r, sem_g, sem_out):
    sid = lax.axis_index("s")
    wid = sid * NC + lax.axis_index("c")
    pbase = wid * PPW

    idx_copies = []
    for b in range(BATCH):
        idx_copies.append(
            pltpu.async_copy(
                ids_hbm.at[pl.ds(b * SEQ + pbase, PPW)],
                idx_v.at[pl.ds(b * PPW, PPW)],
                sem_i,
            )
        )

    pltpu.sync_copy(ptab_hbm.at[pl.ds(pbase, PPW)], pos_sh.at[sid])
    reps = []
    for b in range(BATCH):
        reps.append(
            pltpu.async_copy(
                pos_sh.at[sid],
                rows_v.at[pl.ds(b * PPW, PPW), :],
                sem_r[b],
            )
        )

    for c in idx_copies:
        c.wait()

    gathers = []
    for b in range(BATCH):
        sl = pl.ds(b * PPW, PPW)
        reps[b].wait()
        gathers.append(
            pltpu.async_copy(
                wtab_hbm.at[idx_v.at[sl]],
                rows_v.at[sl, :],
                sem_g[b],
                add=True,
            )
        )

    outs = []
    for b in range(BATCH):
        sl = pl.ds(b * PPW, PPW)
        gathers[b].wait()
        outs.append(
            pltpu.async_copy(
                rows_v.at[sl, :],
                out_hbm.at[pl.ds(b * SEQ + pbase, PPW)],
                sem_out,
            )
        )
    for o in outs:
        o.wait()


def kernel(input_ids, word_embeddings, position_embeddings):
    ids = input_ids.astype(jnp.int32).reshape(-1)
    out = _embed_sum(ids, word_embeddings, position_embeddings)
    return out.reshape(BATCH, SEQ, HIDDEN)
